# baseline (speedup 1.0000x reference)
import functools

import jax
import jax.numpy as jnp
from jax import lax
from jax.experimental import pallas as pl
from jax.experimental.pallas import tpu as pltpu
from jax.experimental.pallas import tpu_sc as plsc


_BLOCK_ROWS = 4000


def _copy_tc_kernel(x_ref, o_ref):
    o_ref[...] = x_ref[...]


def _tc_copy(x):
    n, d = x.shape
    block_rows = min(_BLOCK_ROWS, n)
    grid = ((n + block_rows - 1) // block_rows,)
    spec = pl.BlockSpec((block_rows, d), lambda i: (i, 0))
    return pl.pallas_call(
        _copy_tc_kernel,
        grid=grid,
        in_specs=[spec],
        out_specs=spec,
        out_shape=jax.ShapeDtypeStruct(x.shape, x.dtype),
        compiler_params=pltpu.CompilerParams(
            dimension_semantics=("parallel",),
        ),
    )(x)


def _make_sc_copy(n, d):
    info = plsc.get_sparse_core_info()
    nw = info.num_cores * info.num_subcores
    rows_w = (n // nw) // 8 * 8
    tail = n - rows_w * nw

    mesh = plsc.VectorSubcoreMesh(core_axis_name="c", subcore_axis_name="s")

    @functools.partial(
        pl.kernel,
        mesh=mesh,
        out_type=jax.ShapeDtypeStruct((n, d), jnp.float32),
        scratch_types=[pltpu.VMEM((rows_w, d), jnp.float32)],
    )
    def sc_copy(x_hbm, out_hbm, buf_v):
        wid = lax.axis_index("s") * info.num_cores + lax.axis_index("c")
        base = wid * rows_w
        pltpu.sync_copy(x_hbm.at[pl.ds(base, rows_w)], buf_v)
        pltpu.sync_copy(buf_v, out_hbm.at[pl.ds(base, rows_w)])
        if tail:
            @pl.when(wid == 0)
            def _():
                tbuf = buf_v.at[pl.ds(0, tail)]
                pltpu.sync_copy(x_hbm.at[pl.ds(n - tail, tail)], tbuf)
                pltpu.sync_copy(tbuf, out_hbm.at[pl.ds(n - tail, tail)])

    return sc_copy


def kernel(x_user, x_item, edge_index_user_item, edge_index_item_user):
    del edge_index_user_item, edge_index_item_user
    out_i = _make_sc_copy(*x_item.shape)(x_item)
    out_u = _tc_copy(x_user)
    return (out_u, out_i)


# speedup vs baseline: 2.4707x; 2.4707x over previous
import jax
import jax.numpy as jnp
from jax.experimental import pallas as pl
from jax.experimental.pallas import tpu as pltpu


_BLOCK_ROWS = 4000


def _copy2_kernel(xu_ref, xi_ref, ou_ref, oi_ref):
    ou_ref[...] = xu_ref[...]
    oi_ref[...] = xi_ref[...]


def kernel(x_user, x_item, edge_index_user_item, edge_index_item_user):
    del edge_index_user_item, edge_index_item_user
    n, d = x_user.shape
    block_rows = min(_BLOCK_ROWS, n)
    grid = ((n + block_rows - 1) // block_rows,)
    spec = pl.BlockSpec((block_rows, d), lambda i: (i, 0))
    out_u, out_i = pl.pallas_call(
        _copy2_kernel,
        grid=grid,
        in_specs=[spec, spec],
        out_specs=[spec, spec],
        out_shape=[
            jax.ShapeDtypeStruct(x_user.shape, x_user.dtype),
            jax.ShapeDtypeStruct(x_item.shape, x_item.dtype),
        ],
        compiler_params=pltpu.CompilerParams(
            dimension_semantics=("parallel",),
        ),
    )(x_user, x_item)
    return (out_u, out_i)
